# Initial kernel scaffold; baseline (speedup 1.0000x reference)
import jax
import jax.numpy as jnp
from jax.experimental import pallas as pl


def kernel(x_atoms, x_frags, batch, frag_batch):
    raise NotImplementedError("write your pallas kernel here")


# speedup vs baseline: 2.8418x; 2.8418x over previous
import functools

import jax
import jax.numpy as jnp
from jax import lax
from jax.experimental import pallas as pl
from jax.experimental.pallas import tpu as pltpu
from jax.experimental.pallas import tpu_sc as plsc

NUM_GRAPHS = 10000
N_ATOMS = 320000
N_FRAGS = 100000
EMB = 128

NC = 2
NS = 16

CHUNK = 128

A_CHUNKS = N_ATOMS // CHUNK
F_CHUNKS = N_FRAGS // CHUNK
F_TAIL = N_FRAGS - F_CHUNKS * CHUNK
A_ITERS = (A_CHUNKS + NS - 1) // NS
F_ITERS = (F_CHUNKS + NS - 1) // NS
G_STRIPE = NUM_GRAPHS // NS
OUT_STRIPE = 624
OUT_TAIL = NUM_GRAPHS - NS * OUT_STRIPE


def _body(x_atoms, batch_i, x_frags, frag_i, out, acc, rows, idx, rows_t, idx_t):
    c = lax.axis_index("c")
    s = lax.axis_index("s")

    def zrow(r, _):
        for k in range(EMB // 16):
            rows[r, pl.ds(16 * k, 16)] = jnp.zeros((16,), jnp.float32)
        return 0

    lax.fori_loop(0, 125, zrow, 0)
    for k in range(G_STRIPE // 125):
        pltpu.sync_copy(rows.at[pl.ds(0, 125)],
                        acc.at[pl.ds(s * G_STRIPE + k * 125, 125)])
    plsc.subcore_barrier()

    @pl.when(c == 0)
    def _atoms():
        def step(j, _):
            ci = j * NS + s

            @pl.when(ci < A_CHUNKS)
            def _():
                pltpu.sync_copy(x_atoms.at[pl.ds(ci * CHUNK, CHUNK)], rows)
                pltpu.sync_copy(batch_i.at[pl.ds(ci * CHUNK, CHUNK)], idx)
                pltpu.sync_copy(rows, acc.at[idx], add=True)

            return 0

        lax.fori_loop(0, A_ITERS, step, 0)

    @pl.when(c == 1)
    def _frags():
        def step(j, _):
            ci = j * NS + s

            @pl.when(ci < F_CHUNKS)
            def _():
                pltpu.sync_copy(x_frags.at[pl.ds(ci * CHUNK, CHUNK)], rows)
                pltpu.sync_copy(frag_i.at[pl.ds(ci * CHUNK, CHUNK)], idx)
                pltpu.sync_copy(rows, acc.at[idx], add=True)

            return 0

        lax.fori_loop(0, F_ITERS, step, 0)

        @pl.when(s == NS - 1)
        def _tail():
            base = F_CHUNKS * CHUNK
            pltpu.sync_copy(x_frags.at[pl.ds(base, F_TAIL)], rows_t)
            pltpu.sync_copy(frag_i.at[pl.ds(base, F_TAIL)], idx_t)
            pltpu.sync_copy(rows_t, acc.at[idx_t], add=True)

    plsc.subcore_barrier()

    r0 = s * OUT_STRIPE
    pltpu.sync_copy(acc.at[pl.ds(r0, OUT_STRIPE)],
                    out.at[pl.ds(r0, OUT_STRIPE), pl.ds(c * EMB, EMB)])

    @pl.when(s == NS - 1)
    def _out_tail():
        base = NS * OUT_STRIPE
        pltpu.sync_copy(acc.at[pl.ds(base, OUT_TAIL)],
                        out.at[pl.ds(base, OUT_TAIL), pl.ds(c * EMB, EMB)])


@jax.jit
def _pooled(x_atoms, x_frags, batch_i, frag_i):
    mesh = plsc.VectorSubcoreMesh(core_axis_name="c", subcore_axis_name="s")
    return pl.kernel(
        _body,
        out_type=jax.ShapeDtypeStruct((NUM_GRAPHS, 2 * EMB), jnp.float32),
        mesh=mesh,
        scratch_types=[
            pltpu.VMEM_SHARED((NUM_GRAPHS, EMB), jnp.float32),
            pltpu.VMEM((CHUNK, EMB), jnp.float32),
            pltpu.VMEM((CHUNK,), jnp.int32),
            pltpu.VMEM((F_TAIL, EMB), jnp.float32),
            pltpu.VMEM((F_TAIL,), jnp.int32),
        ],
    )(x_atoms, batch_i, x_frags, frag_i)


def kernel(x_atoms, x_frags, batch, frag_batch):
    return _pooled(x_atoms, x_frags,
                   batch.astype(jnp.int32), frag_batch.astype(jnp.int32))


# speedup vs baseline: 5.3945x; 1.8983x over previous
import functools

import jax
import jax.numpy as jnp
from jax import lax
from jax.experimental import pallas as pl
from jax.experimental.pallas import tpu as pltpu
from jax.experimental.pallas import tpu_sc as plsc

NUM_GRAPHS = 10000
N_ATOMS = 320000
N_FRAGS = 100000
EMB = 128

NC = 2
NS = 16

CHUNK = 128

A_CHUNKS = N_ATOMS // CHUNK
F_CHUNKS = N_FRAGS // CHUNK
F_TAIL = N_FRAGS - F_CHUNKS * CHUNK
A_ITERS = (A_CHUNKS + NS - 1) // NS
F_ITERS = (F_CHUNKS + NS - 1) // NS
G_STRIPE = NUM_GRAPHS // NS
OUT_STRIPE = 624
OUT_TAIL = NUM_GRAPHS - NS * OUT_STRIPE


def _run_array(x_hbm, idx_hbm, nchunks, iters, bufs, acc, s):
    def issue(ci, slot):
        rows_b, idx_b, sem_b = bufs[slot]

        @pl.when(ci < nchunks)
        def _():
            pltpu.async_copy(x_hbm.at[pl.ds(ci * CHUNK, CHUNK)], rows_b, sem_b)
            pltpu.async_copy(idx_hbm.at[pl.ds(ci * CHUNK, CHUNK)], idx_b, sem_b)

    def drain_scatter(ci, slot):
        rows_b, idx_b, sem_b = bufs[slot]

        @pl.when(ci < nchunks)
        def _():
            pltpu.make_async_copy(
                x_hbm.at[pl.ds(ci * CHUNK, CHUNK)], rows_b, sem_b).wait()
            pltpu.make_async_copy(
                idx_hbm.at[pl.ds(ci * CHUNK, CHUNK)], idx_b, sem_b).wait()
            pltpu.sync_copy(rows_b, acc.at[idx_b], add=True)

    issue(s, 0)

    def body(j, _):
        ci = j * NS + s
        ci_next = ci + NS

        @pl.when(j % 2 == 0)
        def _():
            issue(ci_next, 1)
            drain_scatter(ci, 0)

        @pl.when(j % 2 == 1)
        def _():
            issue(ci_next, 0)
            drain_scatter(ci, 1)

        return 0

    lax.fori_loop(0, iters, body, 0)


def _body(x_atoms, batch_i, x_frags, frag_i, out, acc,
          rows, idx, rows2, idx2, sem, sem2, rows_t, idx_t):
    c = lax.axis_index("c")
    s = lax.axis_index("s")
    bufs = ((rows, idx, sem), (rows2, idx2, sem2))

    def zrow(r, _):
        for k in range(EMB // 16):
            rows[r, pl.ds(16 * k, 16)] = jnp.zeros((16,), jnp.float32)
        return 0

    lax.fori_loop(0, 125, zrow, 0)
    for k in range(G_STRIPE // 125):
        pltpu.sync_copy(rows.at[pl.ds(0, 125)],
                        acc.at[pl.ds(s * G_STRIPE + k * 125, 125)])
    plsc.subcore_barrier()

    @pl.when(c == 0)
    def _atoms():
        _run_array(x_atoms, batch_i, A_CHUNKS, A_ITERS, bufs, acc, s)

    @pl.when(c == 1)
    def _frags():
        _run_array(x_frags, frag_i, F_CHUNKS, F_ITERS, bufs, acc, s)

        @pl.when(s == NS - 1)
        def _tail():
            base = F_CHUNKS * CHUNK
            pltpu.sync_copy(x_frags.at[pl.ds(base, F_TAIL)], rows_t)
            pltpu.sync_copy(frag_i.at[pl.ds(base, F_TAIL)], idx_t)
            pltpu.sync_copy(rows_t, acc.at[idx_t], add=True)

    plsc.subcore_barrier()

    r0 = s * OUT_STRIPE
    pltpu.sync_copy(acc.at[pl.ds(r0, OUT_STRIPE)],
                    out.at[pl.ds(r0, OUT_STRIPE), pl.ds(c * EMB, EMB)])

    @pl.when(s == NS - 1)
    def _out_tail():
        base = NS * OUT_STRIPE
        pltpu.sync_copy(acc.at[pl.ds(base, OUT_TAIL)],
                        out.at[pl.ds(base, OUT_TAIL), pl.ds(c * EMB, EMB)])


@jax.jit
def _pooled(x_atoms, x_frags, batch_i, frag_i):
    mesh = plsc.VectorSubcoreMesh(core_axis_name="c", subcore_axis_name="s")
    return pl.kernel(
        _body,
        out_type=jax.ShapeDtypeStruct((NUM_GRAPHS, 2 * EMB), jnp.float32),
        mesh=mesh,
        scratch_types=[
            pltpu.VMEM_SHARED((NUM_GRAPHS, EMB), jnp.float32),
            pltpu.VMEM((CHUNK, EMB), jnp.float32),
            pltpu.VMEM((CHUNK,), jnp.int32),
            pltpu.VMEM((CHUNK, EMB), jnp.float32),
            pltpu.VMEM((CHUNK,), jnp.int32),
            pltpu.SemaphoreType.DMA,
            pltpu.SemaphoreType.DMA,
            pltpu.VMEM((F_TAIL, EMB), jnp.float32),
            pltpu.VMEM((F_TAIL,), jnp.int32),
        ],
    )(x_atoms, batch_i, x_frags, frag_i)


def kernel(x_atoms, x_frags, batch, frag_batch):
    return _pooled(x_atoms, x_frags,
                   batch.astype(jnp.int32), frag_batch.astype(jnp.int32))
